# baseline (speedup 1.0000x reference)
import jax
import jax.numpy as jnp
from jax.experimental import pallas as pl


def kernel(x, pos_emb):
    batch = x.shape[0]
    max_len, d_model = pos_emb.shape
    block_rows = 512

    def body(p_ref, o_ref):
        blk = p_ref[...]
        o_ref[...] = jnp.broadcast_to(blk[None, :, :], (batch, block_rows, d_model))

    return pl.pallas_call(
        body,
        grid=(max_len // block_rows,),
        in_specs=[pl.BlockSpec((block_rows, d_model), lambda i: (i, 0))],
        out_specs=pl.BlockSpec(
            (batch, block_rows, d_model), lambda i: (0, i, 0)
        ),
        out_shape=jax.ShapeDtypeStruct((batch, max_len, d_model), pos_emb.dtype),
    )(pos_emb)


# speedup vs baseline: 1.0191x; 1.0191x over previous
import jax
import jax.numpy as jnp
from jax.experimental import pallas as pl


def kernel(x, pos_emb):
    batch = x.shape[0]
    max_len, d_model = pos_emb.shape
    block_rows = 2048
    bblk = 2

    def body(p_ref, o_ref):
        blk = p_ref[...]
        o_ref[...] = jnp.broadcast_to(blk[None, :, :], (bblk, block_rows, d_model))

    return pl.pallas_call(
        body,
        grid=(max_len // block_rows, batch // bblk),
        in_specs=[pl.BlockSpec((block_rows, d_model), lambda i, b: (i, 0))],
        out_specs=pl.BlockSpec(
            (bblk, block_rows, d_model), lambda i, b: (b, i, 0)
        ),
        out_shape=jax.ShapeDtypeStruct((batch, max_len, d_model), pos_emb.dtype),
    )(pos_emb)


# speedup vs baseline: 1.0307x; 1.0113x over previous
import jax
import jax.numpy as jnp
from jax.experimental import pallas as pl


def kernel(x, pos_emb):
    batch = x.shape[0]
    max_len, d_model = pos_emb.shape
    block_rows = 1024

    def body(p_ref, o_ref):
        blk = p_ref[...]
        o_ref[...] = jnp.broadcast_to(blk[None, :, :], (batch, block_rows, d_model))

    return pl.pallas_call(
        body,
        grid=(max_len // block_rows,),
        in_specs=[pl.BlockSpec((block_rows, d_model), lambda i: (i, 0))],
        out_specs=pl.BlockSpec(
            (batch, block_rows, d_model), lambda i: (0, i, 0)
        ),
        out_shape=jax.ShapeDtypeStruct((batch, max_len, d_model), pos_emb.dtype),
    )(pos_emb)


# speedup vs baseline: 1.0584x; 1.0269x over previous
import jax
import jax.numpy as jnp
from jax.experimental import pallas as pl
from jax.experimental.pallas import tpu as pltpu


def kernel(x, pos_emb):
    batch = x.shape[0]
    max_len, d_model = pos_emb.shape
    block_rows = 1024
    nblk = max_len // block_rows

    def body(p_ref, o_ref, buf, in_sem, out_sem):
        in_copies = [
            pltpu.make_async_copy(
                p_ref.at[pl.ds(i * block_rows, block_rows)],
                buf.at[i],
                in_sem.at[i],
            )
            for i in range(nblk)
        ]
        for c in in_copies:
            c.start()
        out_copies = []
        for i in range(nblk):
            in_copies[i].wait()
            for b in range(batch):
                c = pltpu.make_async_copy(
                    buf.at[i],
                    o_ref.at[b, pl.ds(i * block_rows, block_rows)],
                    out_sem.at[i, b],
                )
                c.start()
                out_copies.append(c)
        for c in out_copies:
            c.wait()

    return pl.pallas_call(
        body,
        in_specs=[pl.BlockSpec(memory_space=pl.ANY)],
        out_specs=pl.BlockSpec(memory_space=pl.ANY),
        out_shape=jax.ShapeDtypeStruct((batch, max_len, d_model), pos_emb.dtype),
        scratch_shapes=[
            pltpu.VMEM((nblk, block_rows, d_model), pos_emb.dtype),
            pltpu.SemaphoreType.DMA((nblk,)),
            pltpu.SemaphoreType.DMA((nblk, batch)),
        ],
    )(pos_emb)


# speedup vs baseline: 1.0712x; 1.0120x over previous
import jax
import jax.numpy as jnp
from jax.experimental import pallas as pl
from jax.experimental.pallas import tpu as pltpu


def kernel(x, pos_emb):
    batch = x.shape[0]
    max_len, d_model = pos_emb.shape
    block_rows = 2048
    nblk = max_len // block_rows

    def body(p_ref, o_ref, buf, in_sem, out_sem):
        in_copies = [
            pltpu.make_async_copy(
                p_ref.at[pl.ds(i * block_rows, block_rows)],
                buf.at[i],
                in_sem.at[i],
            )
            for i in range(nblk)
        ]
        for c in in_copies:
            c.start()
        out_copies = []
        for i in range(nblk):
            in_copies[i].wait()
            for b in range(batch):
                c = pltpu.make_async_copy(
                    buf.at[i],
                    o_ref.at[b, pl.ds(i * block_rows, block_rows)],
                    out_sem.at[i, b],
                )
                c.start()
                out_copies.append(c)
        for c in out_copies:
            c.wait()

    return pl.pallas_call(
        body,
        in_specs=[pl.BlockSpec(memory_space=pl.ANY)],
        out_specs=pl.BlockSpec(memory_space=pl.ANY),
        out_shape=jax.ShapeDtypeStruct((batch, max_len, d_model), pos_emb.dtype),
        scratch_shapes=[
            pltpu.VMEM((nblk, block_rows, d_model), pos_emb.dtype),
            pltpu.SemaphoreType.DMA((nblk,)),
            pltpu.SemaphoreType.DMA((nblk, batch)),
        ],
    )(pos_emb)


# speedup vs baseline: 1.0724x; 1.0012x over previous
import jax
import jax.numpy as jnp
from jax.experimental import pallas as pl
from jax.experimental.pallas import tpu as pltpu


def kernel(x, pos_emb):
    batch = x.shape[0]
    max_len, d_model = pos_emb.shape
    block_rows = 4096
    nblk = max_len // block_rows

    def body(p_ref, o_ref, buf, in_sem, out_sem):
        in_copies = [
            pltpu.make_async_copy(
                p_ref.at[pl.ds(i * block_rows, block_rows)],
                buf.at[i],
                in_sem.at[i],
            )
            for i in range(nblk)
        ]
        for c in in_copies:
            c.start()
        out_copies = []
        for i in range(nblk):
            in_copies[i].wait()
            for b in range(batch):
                c = pltpu.make_async_copy(
                    buf.at[i],
                    o_ref.at[b, pl.ds(i * block_rows, block_rows)],
                    out_sem.at[i, b],
                )
                c.start()
                out_copies.append(c)
        for c in out_copies:
            c.wait()

    return pl.pallas_call(
        body,
        in_specs=[pl.BlockSpec(memory_space=pl.ANY)],
        out_specs=pl.BlockSpec(memory_space=pl.ANY),
        out_shape=jax.ShapeDtypeStruct((batch, max_len, d_model), pos_emb.dtype),
        scratch_shapes=[
            pltpu.VMEM((nblk, block_rows, d_model), pos_emb.dtype),
            pltpu.SemaphoreType.DMA((nblk,)),
            pltpu.SemaphoreType.DMA((nblk, batch)),
        ],
    )(pos_emb)
